# baseline (speedup 1.0000x reference)
import jax
import jax.numpy as jnp
from jax.experimental import pallas as pl
from jax.experimental.pallas import tpu as pltpu

ALPHA = 4
NUM_FRAMES = 32
SLOW_FRAMES = NUM_FRAMES // ALPHA
SLOW_IDX = tuple(
    int(i * (NUM_FRAMES - 1) / (SLOW_FRAMES - 1)) for i in range(SLOW_FRAMES)
)
FRAME_ELEMS = 3 * 224 * 224


def _body(in_hbm, fast_hbm, slow_hbm, copy_sem, gather_sem):
    fast_cp = pltpu.make_async_copy(in_hbm, fast_hbm, copy_sem)
    fast_cp.start()
    slab_cps = []
    for k, s in enumerate(SLOW_IDX):
        cp = pltpu.make_async_copy(in_hbm.at[:, s], slow_hbm.at[:, k], gather_sem)
        cp.start()
        slab_cps.append(cp)
    for cp in slab_cps:
        cp.wait()
    fast_cp.wait()


def kernel(frames):
    b, n, c, h, w = frames.shape
    flat = frames.reshape(b, n, FRAME_ELEMS)
    fast_flat, slow_flat = pl.pallas_call(
        _body,
        in_specs=[pl.BlockSpec(memory_space=pltpu.MemorySpace.HBM)],
        out_specs=[
            pl.BlockSpec(memory_space=pltpu.MemorySpace.HBM),
            pl.BlockSpec(memory_space=pltpu.MemorySpace.HBM),
        ],
        out_shape=[
            jax.ShapeDtypeStruct((b, n, FRAME_ELEMS), frames.dtype),
            jax.ShapeDtypeStruct((b, SLOW_FRAMES, FRAME_ELEMS), frames.dtype),
        ],
        scratch_shapes=[pltpu.SemaphoreType.DMA, pltpu.SemaphoreType.DMA],
    )(flat)
    fast = fast_flat.reshape(b, n, c, h, w)
    slow = slow_flat.reshape(b, SLOW_FRAMES, c, h, w)
    return (slow, fast)


# speedup vs baseline: 10.5156x; 10.5156x over previous
import functools

import jax
import jax.numpy as jnp
from jax import lax
from jax.experimental import pallas as pl
from jax.experimental.pallas import tpu as pltpu
from jax.experimental.pallas import tpu_sc as plsc

ALPHA = 4
NUM_FRAMES = 32
BATCH = 4
SLOW_FRAMES = NUM_FRAMES // ALPHA
FRAME_ELEMS = 3 * 224 * 224

NC = 2
NS = 16
NW = NC * NS

CHUNK = FRAME_ELEMS // 4
CPF = FRAME_ELEMS // CHUNK
FRAMES_PER_W = (BATCH * NUM_FRAMES) // NW
REGION = FRAMES_PER_W * FRAME_ELEMS
NFAST = FRAMES_PER_W * CPF
NJOBS = NFAST + CPF
NBUF = 3

FAST_TOTAL = BATCH * NUM_FRAMES * FRAME_ELEMS
SLOW_TOTAL = BATCH * SLOW_FRAMES * FRAME_ELEMS


def _body(in_hbm, fast_hbm, slow_hbm, *scratch):
    bufs = scratch[:NBUF]
    in_sems = scratch[NBUF : 2 * NBUF]
    out_sems = scratch[2 * NBUF :]
    wid = lax.axis_index("c") * NS + lax.axis_index("s")
    b = wid // SLOW_FRAMES
    k = wid % SLOW_FRAMES
    t_slow = (31 * k) // 7
    base = wid * REGION
    slow_src = (b * NUM_FRAMES + t_slow) * FRAME_ELEMS
    slow_dst = wid * FRAME_ELEMS

    def in_cp(j):
        if j < NFAST:
            src = base + j * CHUNK
        else:
            src = slow_src + (j - NFAST) * CHUNK
        return pltpu.make_async_copy(
            in_hbm.at[pl.ds(src, CHUNK)], bufs[j % NBUF], in_sems[j % NBUF]
        )

    def out_cp(j):
        if j < NFAST:
            dst = fast_hbm.at[pl.ds(base + j * CHUNK, CHUNK)]
        else:
            dst = slow_hbm.at[pl.ds(slow_dst + (j - NFAST) * CHUNK, CHUNK)]
        return pltpu.make_async_copy(bufs[j % NBUF], dst, out_sems[j % NBUF])

    for j in range(NBUF - 1):
        in_cp(j).start()
    for j in range(NJOBS):
        nxt = j + NBUF - 1
        if nxt < NJOBS:
            if j >= 1:
                out_cp(j - 1).wait()
            in_cp(nxt).start()
        in_cp(j).wait()
        out_cp(j).start()
    for j in range(NJOBS - NBUF, NJOBS):
        out_cp(j).wait()


def kernel(frames):
    b, n, c, h, w = frames.shape
    flat = frames.reshape(FAST_TOTAL)
    mesh = plsc.VectorSubcoreMesh(
        core_axis_name="c", subcore_axis_name="s", num_cores=NC, num_subcores=NS
    )
    run = pl.kernel(
        _body,
        out_type=[
            jax.ShapeDtypeStruct((FAST_TOTAL,), frames.dtype),
            jax.ShapeDtypeStruct((SLOW_TOTAL,), frames.dtype),
        ],
        mesh=mesh,
        scratch_types=[pltpu.VMEM((CHUNK,), jnp.float32)] * NBUF
        + [pltpu.SemaphoreType.DMA] * (2 * NBUF),
    )
    fast_flat, slow_flat = run(flat)
    fast = fast_flat.reshape(b, n, c, h, w)
    slow = slow_flat.reshape(b, SLOW_FRAMES, c, h, w)
    return (slow, fast)


# speedup vs baseline: 33.4786x; 3.1837x over previous
import jax
import jax.numpy as jnp
from jax import lax
from jax.experimental import pallas as pl
from jax.experimental.pallas import tpu as pltpu
from jax.experimental.pallas import tpu_sc as plsc

ALPHA = 4
NUM_FRAMES = 32
BATCH = 4
SLOW_FRAMES = NUM_FRAMES // ALPHA

NC = 2
NS = 16
NW = NC * NS

FRAME_ROWS = 3 * 224
LANES = 224
ROWS = 112
CPF = FRAME_ROWS // ROWS
FRAMES_PER_W = (BATCH * NUM_FRAMES) // NW
NFAST = FRAMES_PER_W * CPF
NJOBS = NFAST + CPF
NBUF = 3


def _body(in_hbm, fast_hbm, slow_hbm, *scratch):
    bufs = scratch[:NBUF]
    in_sems = scratch[NBUF : 2 * NBUF]
    out_sems = scratch[2 * NBUF :]
    wid = lax.axis_index("c") * NS + lax.axis_index("s")
    b = wid // SLOW_FRAMES
    k = wid % SLOW_FRAMES
    t_slow = (31 * k) // 7
    base_f = wid * FRAMES_PER_W
    slow_f = b * NUM_FRAMES + t_slow

    def in_cp(j):
        if j < NFAST:
            src = in_hbm.at[base_f + j // CPF, pl.ds((j % CPF) * ROWS, ROWS)]
        else:
            src = in_hbm.at[slow_f, pl.ds((j - NFAST) * ROWS, ROWS)]
        return pltpu.make_async_copy(src, bufs[j % NBUF], in_sems[j % NBUF])

    def out_cp(j):
        if j < NFAST:
            dst = fast_hbm.at[base_f + j // CPF, pl.ds((j % CPF) * ROWS, ROWS)]
        else:
            dst = slow_hbm.at[wid, pl.ds((j - NFAST) * ROWS, ROWS)]
        return pltpu.make_async_copy(bufs[j % NBUF], dst, out_sems[j % NBUF])

    for j in range(NBUF - 1):
        in_cp(j).start()
    for j in range(NJOBS):
        nxt = j + NBUF - 1
        if nxt < NJOBS:
            if j >= 1:
                out_cp(j - 1).wait()
            in_cp(nxt).start()
        in_cp(j).wait()
        out_cp(j).start()
    for j in range(NJOBS - NBUF, NJOBS):
        out_cp(j).wait()


def kernel(frames):
    b, n, c, h, w = frames.shape
    flat = frames.reshape(b * n, FRAME_ROWS, LANES)
    mesh = plsc.VectorSubcoreMesh(
        core_axis_name="c", subcore_axis_name="s", num_cores=NC, num_subcores=NS
    )
    run = pl.kernel(
        _body,
        out_type=[
            jax.ShapeDtypeStruct((b * n, FRAME_ROWS, LANES), frames.dtype),
            jax.ShapeDtypeStruct((b * SLOW_FRAMES, FRAME_ROWS, LANES), frames.dtype),
        ],
        mesh=mesh,
        scratch_types=[pltpu.VMEM((ROWS, LANES), jnp.float32)] * NBUF
        + [pltpu.SemaphoreType.DMA] * (2 * NBUF),
        compiler_params=pltpu.CompilerParams(use_tc_tiling_on_sc=True),
    )
    fast_flat, slow_flat = run(flat)
    fast = fast_flat.reshape(b, n, c, h, w)
    slow = slow_flat.reshape(b, SLOW_FRAMES, c, h, w)
    return (slow, fast)
